# Initial kernel scaffold; baseline (speedup 1.0000x reference)
import jax
import jax.numpy as jnp
from jax.experimental import pallas as pl


def kernel(state, obs, log_em):
    raise NotImplementedError("write your pallas kernel here")


# speedup vs baseline: 241.2262x; 241.2262x over previous
import functools

import jax
import jax.numpy as jnp
from jax import lax
from jax.experimental import pallas as pl
from jax.experimental.pallas import tpu as pltpu
from jax.experimental.pallas import tpu_sc as plsc

N_STATES = 2
N_OBVS = 6
LANES = 16
NUM_CORES = 2
NUM_SUBCORES = 16
NUM_WORKERS = NUM_CORES * NUM_SUBCORES
CHUNK = 25600


def _make_sc_lookup(total: int):
    per_worker = total // NUM_WORKERS
    n_chunks = per_worker // CHUNK
    assert per_worker * NUM_WORKERS == total
    assert n_chunks * CHUNK == per_worker

    mesh = plsc.VectorSubcoreMesh(core_axis_name="c", subcore_axis_name="s")

    @functools.partial(
        pl.kernel,
        mesh=mesh,
        out_type=jax.ShapeDtypeStruct((total,), jnp.float32),
        compiler_params=pltpu.CompilerParams(needs_layout_passes=False),
        scratch_types=[
            pltpu.VMEM((CHUNK,), jnp.int32),
            pltpu.VMEM((CHUNK,), jnp.int32),
            pltpu.VMEM((CHUNK,), jnp.float32),
            pltpu.VMEM((LANES,), jnp.float32),
        ],
    )
    def sc_lookup(s_hbm, o_hbm, em_hbm, out_hbm, s_v, o_v, out_v, em_v):
        wid = lax.axis_index("s") * NUM_CORES + lax.axis_index("c")
        base = wid * per_worker
        pltpu.sync_copy(em_hbm, em_v)

        def chunk_body(ci, carry):
            off = base + ci * CHUNK
            pltpu.sync_copy(s_hbm.at[pl.ds(off, CHUNK)], s_v)
            pltpu.sync_copy(o_hbm.at[pl.ds(off, CHUNK)], o_v)

            def vec_body(i, c):
                s = s_v[pl.ds(i * LANES, LANES)]
                o = o_v[pl.ds(i * LANES, LANES)]
                flat = s * N_OBVS + o
                out_v[pl.ds(i * LANES, LANES)] = plsc.load_gather(em_v, [flat])
                return c

            lax.fori_loop(0, CHUNK // LANES, vec_body, 0)
            pltpu.sync_copy(out_v, out_hbm.at[pl.ds(off, CHUNK)])
            return carry

        lax.fori_loop(0, n_chunks, chunk_body, 0)

    return sc_lookup


def kernel(state, obs, log_em):
    batch, hist = state.shape
    total = batch * hist
    s_flat = state.reshape(total)
    o_flat = obs.reshape(total)
    em_pad = jnp.pad(log_em.reshape(N_STATES * N_OBVS), (0, LANES - N_STATES * N_OBVS))
    out_flat = _make_sc_lookup(total)(s_flat, o_flat, em_pad)
    return out_flat.reshape(batch, hist)


# speedup vs baseline: 282.5760x; 1.1714x over previous
import functools

import jax
import jax.numpy as jnp
from jax import lax
from jax.experimental import pallas as pl
from jax.experimental.pallas import tpu as pltpu
from jax.experimental.pallas import tpu_sc as plsc

N_STATES = 2
N_OBVS = 6
LANES = 16
NUM_CORES = 2
NUM_SUBCORES = 16
NUM_WORKERS = NUM_CORES * NUM_SUBCORES
CHUNK = 12800
NBUF = 2


def _make_sc_lookup(total: int):
    per_worker = total // NUM_WORKERS
    n_chunks = per_worker // CHUNK
    assert per_worker * NUM_WORKERS == total
    assert n_chunks * CHUNK == per_worker and n_chunks >= NBUF

    mesh = plsc.VectorSubcoreMesh(core_axis_name="c", subcore_axis_name="s")

    @functools.partial(
        pl.kernel,
        mesh=mesh,
        out_type=jax.ShapeDtypeStruct((total,), jnp.float32),
        compiler_params=pltpu.CompilerParams(needs_layout_passes=False),
        scratch_types=(
            [pltpu.VMEM((CHUNK,), jnp.int32) for _ in range(2 * NBUF)]
            + [pltpu.VMEM((CHUNK,), jnp.float32) for _ in range(NBUF)]
            + [pltpu.VMEM((LANES,), jnp.float32)]
            + [pltpu.SemaphoreType.DMA for _ in range(2 * NBUF)]
        ),
    )
    def sc_lookup(s_hbm, o_hbm, em_hbm, out_hbm, *scratch):
        s_v = scratch[0:NBUF]
        o_v = scratch[NBUF : 2 * NBUF]
        out_v = scratch[2 * NBUF : 3 * NBUF]
        em_v = scratch[3 * NBUF]
        sem_in = scratch[3 * NBUF + 1 : 3 * NBUF + 1 + NBUF]
        sem_out = scratch[3 * NBUF + 1 + NBUF :]

        wid = lax.axis_index("s") * NUM_CORES + lax.axis_index("c")
        base = wid * per_worker
        pltpu.sync_copy(em_hbm, em_v)

        def start_in(g):
            off = base + g * CHUNK
            b = g % NBUF
            return (
                pltpu.async_copy(s_hbm.at[pl.ds(off, CHUNK)], s_v[b], sem_in[b]),
                pltpu.async_copy(o_hbm.at[pl.ds(off, CHUNK)], o_v[b], sem_in[b]),
            )

        in_cp = {0: start_in(0)}
        out_cp = {}
        for g in range(n_chunks):
            b = g % NBUF
            if g + 1 < n_chunks:
                in_cp[g + 1] = start_in(g + 1)
            for cp in in_cp.pop(g):
                cp.wait()
            if g >= NBUF:
                out_cp.pop(g - NBUF).wait()

            sb, ob, ub = s_v[b], o_v[b], out_v[b]

            @plsc.parallel_loop(0, CHUNK, LANES, unroll=8)
            def body(i):
                flat = sb[pl.ds(i, LANES)] * N_OBVS + ob[pl.ds(i, LANES)]
                ub[pl.ds(i, LANES)] = plsc.load_gather(em_v, [flat])

            off = base + g * CHUNK
            out_cp[g] = pltpu.async_copy(
                ub, out_hbm.at[pl.ds(off, CHUNK)], sem_out[b]
            )
        for g in sorted(out_cp):
            out_cp.pop(g).wait()

    return sc_lookup


def kernel(state, obs, log_em):
    batch, hist = state.shape
    total = batch * hist
    s_flat = state.reshape(total)
    o_flat = obs.reshape(total)
    em_pad = jnp.pad(log_em.reshape(N_STATES * N_OBVS), (0, LANES - N_STATES * N_OBVS))
    out_flat = _make_sc_lookup(total)(s_flat, o_flat, em_pad)
    return out_flat.reshape(batch, hist)


# speedup vs baseline: 470.6641x; 1.6656x over previous
import functools

import jax
import jax.numpy as jnp
from jax import lax
from jax.experimental import pallas as pl
from jax.experimental.pallas import tpu as pltpu
from jax.experimental.pallas import tpu_sc as plsc

N_STATES = 2
N_OBVS = 6
LANES = 16
NUM_CORES = 2
NUM_SUBCORES = 16
NUM_WORKERS = NUM_CORES * NUM_SUBCORES
CHUNK_ROWS = 64
NBUF = 2


def _make_sc_lookup(n_rows: int, n_cols: int):
    per_worker = n_rows // NUM_WORKERS
    n_chunks = per_worker // CHUNK_ROWS
    assert per_worker * NUM_WORKERS == n_rows
    assert n_chunks * CHUNK_ROWS == per_worker and n_chunks >= NBUF
    strips = list(range(0, n_cols - LANES + 1, LANES))
    if strips[-1] + LANES < n_cols:
        strips.append(n_cols - LANES)

    mesh = plsc.VectorSubcoreMesh(core_axis_name="c", subcore_axis_name="s")

    @functools.partial(
        pl.kernel,
        mesh=mesh,
        out_type=jax.ShapeDtypeStruct((n_rows, n_cols), jnp.float32),
        compiler_params=pltpu.CompilerParams(needs_layout_passes=False),
        scratch_types=(
            [pltpu.VMEM((CHUNK_ROWS, n_cols), jnp.int32) for _ in range(2 * NBUF)]
            + [pltpu.VMEM((CHUNK_ROWS, n_cols), jnp.float32) for _ in range(NBUF)]
            + [pltpu.VMEM((LANES,), jnp.float32)]
            + [pltpu.SemaphoreType.DMA for _ in range(2 * NBUF)]
        ),
    )
    def sc_lookup(s_hbm, o_hbm, em_hbm, out_hbm, *scratch):
        s_v = scratch[0:NBUF]
        o_v = scratch[NBUF : 2 * NBUF]
        out_v = scratch[2 * NBUF : 3 * NBUF]
        em_v = scratch[3 * NBUF]
        sem_in = scratch[3 * NBUF + 1 : 3 * NBUF + 1 + NBUF]
        sem_out = scratch[3 * NBUF + 1 + NBUF :]

        wid = lax.axis_index("s") * NUM_CORES + lax.axis_index("c")
        base = wid * per_worker
        pltpu.sync_copy(em_hbm, em_v)

        def start_in(g):
            off = base + g * CHUNK_ROWS
            b = g % NBUF
            return (
                pltpu.async_copy(
                    s_hbm.at[pl.ds(off, CHUNK_ROWS), :], s_v[b], sem_in[b]
                ),
                pltpu.async_copy(
                    o_hbm.at[pl.ds(off, CHUNK_ROWS), :], o_v[b], sem_in[b]
                ),
            )

        in_cp = {0: start_in(0)}
        out_cp = {}
        for g in range(n_chunks):
            b = g % NBUF
            if g + 1 < n_chunks:
                in_cp[g + 1] = start_in(g + 1)
            for cp in in_cp.pop(g):
                cp.wait()
            if g >= NBUF:
                out_cp.pop(g - NBUF).wait()

            sb, ob, ub = s_v[b], o_v[b], out_v[b]

            @plsc.parallel_loop(0, CHUNK_ROWS, 1, unroll=2)
            def body(r):
                for c in strips:
                    flat = sb[r, pl.ds(c, LANES)] * N_OBVS + ob[r, pl.ds(c, LANES)]
                    ub[r, pl.ds(c, LANES)] = plsc.load_gather(em_v, [flat])

            off = base + g * CHUNK_ROWS
            out_cp[g] = pltpu.async_copy(
                ub, out_hbm.at[pl.ds(off, CHUNK_ROWS), :], sem_out[b]
            )
        for g in sorted(out_cp):
            out_cp.pop(g).wait()

    return sc_lookup


def kernel(state, obs, log_em):
    n_rows, n_cols = state.shape
    em_pad = jnp.pad(log_em.reshape(N_STATES * N_OBVS), (0, LANES - N_STATES * N_OBVS))
    return _make_sc_lookup(n_rows, n_cols)(state, obs, em_pad)


# speedup vs baseline: 868.0681x; 1.8443x over previous
import functools

import jax
import jax.numpy as jnp
from jax import lax
from jax.experimental import pallas as pl
from jax.experimental.pallas import tpu as pltpu
from jax.experimental.pallas import tpu_sc as plsc

N_STATES = 2
N_OBVS = 6
LANES = 16
NUM_CORES = 2
NUM_SUBCORES = 16
NUM_WORKERS = NUM_CORES * NUM_SUBCORES
CHUNK_ROWS = 8
NBUF = 2


def _make_sc_lookup(n_rows: int, n_cols: int):
    cols_per_worker = n_cols // NUM_WORKERS
    n_chunks = n_rows // CHUNK_ROWS
    assert cols_per_worker * NUM_WORKERS == n_cols
    assert cols_per_worker % LANES == 0
    assert n_chunks * CHUNK_ROWS == n_rows
    assert n_chunks >= 5 and (n_chunks - 5) % 2 == 0
    strips = CHUNK_ROWS * (cols_per_worker // LANES)
    sh = (CHUNK_ROWS, cols_per_worker)

    mesh = plsc.VectorSubcoreMesh(core_axis_name="c", subcore_axis_name="s")

    @functools.partial(
        pl.kernel,
        mesh=mesh,
        out_type=jax.ShapeDtypeStruct((n_rows, n_cols), jnp.float32),
        compiler_params=pltpu.CompilerParams(needs_layout_passes=False),
        scratch_types=(
            [pltpu.VMEM(sh, jnp.int32) for _ in range(2 * NBUF)]
            + [pltpu.VMEM(sh, jnp.float32) for _ in range(NBUF)]
            + [pltpu.VMEM((N_STATES, N_OBVS), jnp.float32)]
            + [pltpu.SemaphoreType.DMA for _ in range(2 * NBUF)]
        ),
    )
    def sc_lookup(s_hbm, o_hbm, em_hbm, out_hbm, *scratch):
        s_v = scratch[0:NBUF]
        o_v = scratch[NBUF : 2 * NBUF]
        out_v = scratch[2 * NBUF : 3 * NBUF]
        em_v = scratch[3 * NBUF]
        sem_in = scratch[3 * NBUF + 1 : 3 * NBUF + 1 + NBUF]
        sem_out = scratch[3 * NBUF + 1 + NBUF :]

        wid = lax.axis_index("s") * NUM_CORES + lax.axis_index("c")
        col0 = wid * cols_per_worker
        cols = pl.ds(col0, cols_per_worker)
        pltpu.sync_copy(em_hbm, em_v)

        def rows_of(g):
            return pl.ds(g * CHUNK_ROWS, CHUNK_ROWS)

        def start_in(g, b):
            pltpu.async_copy(s_hbm.at[rows_of(g), cols], s_v[b], sem_in[b])
            pltpu.async_copy(o_hbm.at[rows_of(g), cols], o_v[b], sem_in[b])

        def wait_in(g, b):
            pltpu.make_async_copy(s_hbm.at[rows_of(g), cols], s_v[b], sem_in[b]).wait()
            pltpu.make_async_copy(o_hbm.at[rows_of(g), cols], o_v[b], sem_in[b]).wait()

        def start_out(g, b):
            pltpu.async_copy(out_v[b], out_hbm.at[rows_of(g), cols], sem_out[b])

        def wait_out(g, b):
            pltpu.make_async_copy(
                out_v[b], out_hbm.at[rows_of(g), cols], sem_out[b]
            ).wait()

        def compute(b):
            sb, ob, ub = s_v[b], o_v[b], out_v[b]
            spr = cols_per_worker // LANES

            @plsc.parallel_loop(0, strips, 1, unroll=8)
            def body(t):
                r = t // spr
                c = (t % spr) * LANES
                s = sb[r, pl.ds(c, LANES)]
                o = ob[r, pl.ds(c, LANES)]
                ub[r, pl.ds(c, LANES)] = plsc.load_gather(em_v, [s, o])

        start_in(0, 0)
        start_in(1, 1)
        wait_in(0, 0)
        compute(0)
        start_out(0, 0)
        start_in(2, 0)
        wait_in(1, 1)
        compute(1)
        start_out(1, 1)
        start_in(3, 1)
        wait_in(2, 0)
        wait_out(0, 0)
        compute(0)
        start_out(2, 0)

        def pair(k, carry):
            g1 = 3 + 2 * k
            start_in(g1 + 1, 0)
            wait_in(g1, 1)
            wait_out(g1 - 2, 1)
            compute(1)
            start_out(g1, 1)
            g2 = g1 + 1
            start_in(g2 + 1, 1)
            wait_in(g2, 0)
            wait_out(g2 - 2, 0)
            compute(0)
            start_out(g2, 0)
            return carry

        n_pairs = (n_chunks - 5) // 2
        lax.fori_loop(0, n_pairs, pair, 0)

        gA = 3 + 2 * n_pairs
        start_in(gA + 1, 0)
        wait_in(gA, 1)
        wait_out(gA - 2, 1)
        compute(1)
        start_out(gA, 1)
        gB = gA + 1
        wait_in(gB, 0)
        wait_out(gB - 2, 0)
        compute(0)
        start_out(gB, 0)
        wait_out(gA, 1)
        wait_out(gB, 0)

    return sc_lookup


def kernel(state, obs, log_em):
    n_rows, n_cols = state.shape
    out_t = _make_sc_lookup(n_cols, n_rows)(state.T, obs.T, log_em)
    return out_t.T


# speedup vs baseline: 894.6730x; 1.0306x over previous
import functools

import jax
import jax.numpy as jnp
from jax import lax
from jax.experimental import pallas as pl
from jax.experimental.pallas import tpu as pltpu
from jax.experimental.pallas import tpu_sc as plsc

N_STATES = 2
N_OBVS = 6
LANES = 16
NUM_CORES = 2
NUM_SUBCORES = 16
NUM_WORKERS = NUM_CORES * NUM_SUBCORES
CHUNK_ROWS = 40
NBUF = 2


def _make_sc_lookup(n_rows: int, n_cols: int):
    cols_per_worker = n_cols // NUM_WORKERS
    n_chunks = n_rows // CHUNK_ROWS
    assert cols_per_worker * NUM_WORKERS == n_cols
    assert cols_per_worker % LANES == 0
    assert n_chunks * CHUNK_ROWS == n_rows and n_chunks >= NBUF
    sh = (CHUNK_ROWS, cols_per_worker)

    mesh = plsc.VectorSubcoreMesh(core_axis_name="c", subcore_axis_name="s")

    @functools.partial(
        pl.kernel,
        mesh=mesh,
        out_type=jax.ShapeDtypeStruct((n_rows, n_cols), jnp.float32),
        compiler_params=pltpu.CompilerParams(needs_layout_passes=False),
        scratch_types=(
            [pltpu.VMEM(sh, jnp.int32) for _ in range(2 * NBUF)]
            + [pltpu.VMEM(sh, jnp.float32) for _ in range(NBUF)]
            + [pltpu.VMEM((N_STATES, N_OBVS), jnp.float32)]
            + [pltpu.SemaphoreType.DMA for _ in range(2 * NBUF)]
        ),
    )
    def sc_lookup(s_hbm, o_hbm, em_hbm, out_hbm, *scratch):
        s_v = scratch[0:NBUF]
        o_v = scratch[NBUF : 2 * NBUF]
        out_v = scratch[2 * NBUF : 3 * NBUF]
        em_v = scratch[3 * NBUF]
        sem_in = scratch[3 * NBUF + 1 : 3 * NBUF + 1 + NBUF]
        sem_out = scratch[3 * NBUF + 1 + NBUF :]

        wid = lax.axis_index("s") * NUM_CORES + lax.axis_index("c")
        cols = pl.ds(wid * cols_per_worker, cols_per_worker)
        pltpu.sync_copy(em_hbm, em_v)

        def start_in(g):
            b = g % NBUF
            rows = pl.ds(g * CHUNK_ROWS, CHUNK_ROWS)
            return (
                pltpu.async_copy(s_hbm.at[rows, cols], s_v[b], sem_in[b]),
                pltpu.async_copy(o_hbm.at[rows, cols], o_v[b], sem_in[b]),
            )

        in_cp = {0: start_in(0)}
        out_cp = {}
        for g in range(n_chunks):
            b = g % NBUF
            if g + 1 < n_chunks:
                in_cp[g + 1] = start_in(g + 1)
            for cp in in_cp.pop(g):
                cp.wait()
            if g >= NBUF:
                out_cp.pop(g - NBUF).wait()

            sb, ob, ub = s_v[b], o_v[b], out_v[b]
            spr = cols_per_worker // LANES

            @plsc.parallel_loop(0, CHUNK_ROWS * spr, 1, unroll=8)
            def body(t):
                r = t // spr
                c = (t % spr) * LANES
                s = sb[r, pl.ds(c, LANES)]
                o = ob[r, pl.ds(c, LANES)]
                ub[r, pl.ds(c, LANES)] = plsc.load_gather(em_v, [s, o])

            out_cp[g] = pltpu.async_copy(
                ub, out_hbm.at[pl.ds(g * CHUNK_ROWS, CHUNK_ROWS), cols], sem_out[b]
            )
        for g in sorted(out_cp):
            out_cp.pop(g).wait()

    return sc_lookup


def kernel(state, obs, log_em):
    n_rows, n_cols = state.shape
    out_t = _make_sc_lookup(n_cols, n_rows)(state.T, obs.T, log_em)
    return out_t.T


# speedup vs baseline: 1043.9400x; 1.1668x over previous
import functools

import jax
import jax.numpy as jnp
from jax import lax
from jax.experimental import pallas as pl
from jax.experimental.pallas import tpu as pltpu
from jax.experimental.pallas import tpu_sc as plsc

N_STATES = 2
N_OBVS = 6
LANES = 16
NUM_CORES = 2
NUM_SUBCORES = 16
NUM_WORKERS = NUM_CORES * NUM_SUBCORES
CHUNK_ROWS = 40
NBUF = 2


def _make_sc_lookup(n_rows: int, n_cols: int):
    cols_per_worker = n_cols // NUM_WORKERS
    n_chunks = n_rows // CHUNK_ROWS
    assert cols_per_worker * NUM_WORKERS == n_cols
    assert cols_per_worker % LANES == 0
    assert n_chunks * CHUNK_ROWS == n_rows and n_chunks >= NBUF
    sh = (CHUNK_ROWS, cols_per_worker)

    mesh = plsc.VectorSubcoreMesh(core_axis_name="c", subcore_axis_name="s")

    @functools.partial(
        pl.kernel,
        mesh=mesh,
        out_type=jax.ShapeDtypeStruct((n_rows, n_cols), jnp.float32),
        compiler_params=pltpu.CompilerParams(needs_layout_passes=False),
        scratch_types=(
            [pltpu.VMEM(sh, jnp.int32) for _ in range(2 * NBUF)]
            + [pltpu.VMEM(sh, jnp.float32) for _ in range(NBUF)]
            + [pltpu.VMEM((LANES,), jnp.float32)]
            + [pltpu.SemaphoreType.DMA for _ in range(2 * NBUF)]
        ),
    )
    def sc_lookup(s_hbm, o_hbm, em_hbm, out_hbm, *scratch):
        s_v = scratch[0:NBUF]
        o_v = scratch[NBUF : 2 * NBUF]
        out_v = scratch[2 * NBUF : 3 * NBUF]
        em_v = scratch[3 * NBUF]
        sem_in = scratch[3 * NBUF + 1 : 3 * NBUF + 1 + NBUF]
        sem_out = scratch[3 * NBUF + 1 + NBUF :]

        wid = lax.axis_index("s") * NUM_CORES + lax.axis_index("c")
        cols = pl.ds(wid * cols_per_worker, cols_per_worker)
        pltpu.sync_copy(em_hbm, em_v)

        def start_in(g):
            b = g % NBUF
            rows = pl.ds(g * CHUNK_ROWS, CHUNK_ROWS)
            return (
                pltpu.async_copy(s_hbm.at[rows, cols], s_v[b], sem_in[b]),
                pltpu.async_copy(o_hbm.at[rows, cols], o_v[b], sem_in[b]),
            )

        in_cp = {0: start_in(0)}
        out_cp = {}
        for g in range(n_chunks):
            b = g % NBUF
            if g + 1 < n_chunks:
                in_cp[g + 1] = start_in(g + 1)
            for cp in in_cp.pop(g):
                cp.wait()
            if g >= NBUF:
                out_cp.pop(g - NBUF).wait()

            sb, ob, ub = s_v[b], o_v[b], out_v[b]
            spr = cols_per_worker // LANES

            @plsc.parallel_loop(0, CHUNK_ROWS * spr, 1, unroll=8)
            def body(t):
                r = t // spr
                c = (t % spr) * LANES
                flat = sb[r, pl.ds(c, LANES)] * N_OBVS + ob[r, pl.ds(c, LANES)]
                ub[r, pl.ds(c, LANES)] = plsc.load_gather(em_v, [flat])

            out_cp[g] = pltpu.async_copy(
                ub, out_hbm.at[pl.ds(g * CHUNK_ROWS, CHUNK_ROWS), cols], sem_out[b]
            )
        for g in sorted(out_cp):
            out_cp.pop(g).wait()

    return sc_lookup


def kernel(state, obs, log_em):
    n_rows, n_cols = state.shape
    em_pad = jnp.pad(log_em.reshape(N_STATES * N_OBVS), (0, LANES - N_STATES * N_OBVS))
    out_t = _make_sc_lookup(n_cols, n_rows)(state.T, obs.T, em_pad)
    return out_t.T


# speedup vs baseline: 1082.0357x; 1.0365x over previous
import functools

import jax
import jax.numpy as jnp
from jax import lax
from jax.experimental import pallas as pl
from jax.experimental.pallas import tpu as pltpu
from jax.experimental.pallas import tpu_sc as plsc

N_STATES = 2
N_OBVS = 6
LANES = 16
NUM_CORES = 2
NUM_SUBCORES = 16
NUM_WORKERS = NUM_CORES * NUM_SUBCORES
CHUNK_ROWS = 40
NBUF = 2


def _make_sc_lookup(n_rows: int, n_cols: int):
    cols_per_worker = n_cols // NUM_WORKERS
    n_chunks = n_rows // CHUNK_ROWS
    assert cols_per_worker * NUM_WORKERS == n_cols
    assert cols_per_worker % LANES == 0
    assert n_chunks * CHUNK_ROWS == n_rows and n_chunks >= NBUF
    sh = (CHUNK_ROWS, cols_per_worker)

    mesh = plsc.VectorSubcoreMesh(core_axis_name="c", subcore_axis_name="s")

    @functools.partial(
        pl.kernel,
        mesh=mesh,
        out_type=jax.ShapeDtypeStruct((n_rows, n_cols), jnp.float32),
        compiler_params=pltpu.CompilerParams(needs_layout_passes=False),
        scratch_types=(
            [pltpu.VMEM(sh, jnp.int32) for _ in range(2 * NBUF)]
            + [pltpu.VMEM(sh, jnp.float32) for _ in range(NBUF)]
            + [pltpu.VMEM((LANES,), jnp.float32)]
            + [pltpu.SemaphoreType.DMA for _ in range(2 * NBUF)]
        ),
    )
    def sc_lookup(s_hbm, o_hbm, em_hbm, out_hbm, *scratch):
        s_v = scratch[0:NBUF]
        o_v = scratch[NBUF : 2 * NBUF]
        out_v = scratch[2 * NBUF : 3 * NBUF]
        em_v = scratch[3 * NBUF]
        sem_in = scratch[3 * NBUF + 1 : 3 * NBUF + 1 + NBUF]
        sem_out = scratch[3 * NBUF + 1 + NBUF :]

        wid = lax.axis_index("s") * NUM_CORES + lax.axis_index("c")
        cols = pl.ds(wid * cols_per_worker, cols_per_worker)

        def start_in(g):
            b = g % NBUF
            rows = pl.ds(g * CHUNK_ROWS, CHUNK_ROWS)
            return (
                pltpu.async_copy(s_hbm.at[rows, cols], s_v[b], sem_in[b]),
                pltpu.async_copy(o_hbm.at[rows, cols], o_v[b], sem_in[b]),
            )

        in_cp = {0: start_in(0)}
        pltpu.sync_copy(em_hbm, em_v)
        out_cp = {}
        for g in range(n_chunks):
            b = g % NBUF
            if g + 1 < n_chunks:
                in_cp[g + 1] = start_in(g + 1)
            for cp in in_cp.pop(g):
                cp.wait()
            if g >= NBUF:
                out_cp.pop(g - NBUF).wait()

            sb, ob, ub = s_v[b], o_v[b], out_v[b]
            spr = cols_per_worker // LANES

            @plsc.parallel_loop(0, CHUNK_ROWS * spr, 1, unroll=8)
            def body(t):
                r = t // spr
                c = (t % spr) * LANES
                flat = sb[r, pl.ds(c, LANES)] * N_OBVS + ob[r, pl.ds(c, LANES)]
                ub[r, pl.ds(c, LANES)] = plsc.load_gather(em_v, [flat])

            out_cp[g] = pltpu.async_copy(
                ub, out_hbm.at[pl.ds(g * CHUNK_ROWS, CHUNK_ROWS), cols], sem_out[b]
            )
        for g in sorted(out_cp):
            out_cp.pop(g).wait()

    return sc_lookup


def kernel(state, obs, log_em):
    n_rows, n_cols = state.shape
    em_pad = jnp.pad(log_em.reshape(N_STATES * N_OBVS), (0, LANES - N_STATES * N_OBVS))
    out_t = _make_sc_lookup(n_cols, n_rows)(state.T, obs.T, em_pad)
    return out_t.T
